# baseline (speedup 1.0000x reference)
import jax
import jax.numpy as jnp
from jax.experimental import pallas as pl
from jax.experimental.pallas import tpu as pltpu

_N = 3_200_000
_STEPS = 25
_BK = _N // _STEPS


def _body(s_ref, m_ref, o_ref, acc_ref):
    i = pl.program_id(0)

    @pl.when(i == 0)
    def _():
        acc_ref[...] = jnp.zeros((8, 128), jnp.float32)

    s2 = s_ref[...].reshape(_BK // 128, 128)
    m2 = m_ref[...].reshape(_BK // 128, 128)
    x = jnp.where(m2 != 0, s2, 0.0)
    acc_ref[...] += x.reshape(_BK // 1024, 8, 128).sum(axis=0)

    @pl.when(i == _STEPS - 1)
    def _():
        o_ref[0] = jnp.sum(acc_ref[...]) * (1.0 / _N)


def kernel(scores, mask):
    out = pl.pallas_call(
        _body,
        grid=(_STEPS,),
        in_specs=[
            pl.BlockSpec((_BK,), lambda i: (i,)),
            pl.BlockSpec((_BK,), lambda i: (i,)),
        ],
        out_specs=pl.BlockSpec((1,), lambda i: (0,), memory_space=pltpu.SMEM),
        out_shape=jax.ShapeDtypeStruct((1,), jnp.float32),
        scratch_shapes=[pltpu.VMEM((8, 128), jnp.float32)],
    )(scores, mask.view(jnp.int8))
    return out[0]


# speedup vs baseline: 1.2921x; 1.2921x over previous
import jax
import jax.numpy as jnp
from jax.experimental import pallas as pl
from jax.experimental.pallas import tpu as pltpu

_N = 3_200_000
_ROWS = _N // 128
_STEPS = 5
_BR = _ROWS // _STEPS


def _body(s_ref, m_ref, o_ref, acc_ref):
    i = pl.program_id(0)

    @pl.when(i == 0)
    def _():
        acc_ref[...] = jnp.zeros((8, 128), jnp.float32)

    x = jnp.where(m_ref[...], s_ref[...], 0.0)
    acc_ref[...] += x.reshape(_BR // 8, 8, 128).sum(axis=0)

    @pl.when(i == _STEPS - 1)
    def _():
        o_ref[0] = jnp.sum(acc_ref[...]) * (1.0 / _N)


def kernel(scores, mask):
    s2 = scores.reshape(_ROWS, 128)
    m2 = mask.reshape(_ROWS, 128)
    out = pl.pallas_call(
        _body,
        grid=(_STEPS,),
        in_specs=[
            pl.BlockSpec((_BR, 128), lambda i: (i, 0)),
            pl.BlockSpec((_BR, 128), lambda i: (i, 0)),
        ],
        out_specs=pl.BlockSpec((1,), lambda i: (0,), memory_space=pltpu.SMEM),
        out_shape=jax.ShapeDtypeStruct((1,), jnp.float32),
        scratch_shapes=[pltpu.VMEM((8, 128), jnp.float32)],
    )(s2, m2)
    return out[0]


# speedup vs baseline: 1.6147x; 1.2497x over previous
import jax
import jax.numpy as jnp
from jax.experimental import pallas as pl
from jax.experimental.pallas import tpu as pltpu

_N = 3_200_000
_STEPS = 5
_BK = _N // _STEPS
_BR = _BK // 128


def _body(s_ref, m_ref, o_ref, acc_ref):
    i = pl.program_id(0)

    @pl.when(i == 0)
    def _():
        acc_ref[...] = jnp.zeros((8, 128), jnp.float32)

    s2 = s_ref[...]
    m2 = m_ref[...].reshape(_BR, 128)
    x = jnp.where(m2 != 0, s2, 0.0)
    acc_ref[...] += x.reshape(_BR // 8, 8, 128).sum(axis=0)

    @pl.when(i == _STEPS - 1)
    def _():
        o_ref[0] = jnp.sum(acc_ref[...]) * (1.0 / _N)


def kernel(scores, mask):
    s2 = scores.reshape(_N // 128, 128)
    out = pl.pallas_call(
        _body,
        grid=(_STEPS,),
        in_specs=[
            pl.BlockSpec((_BR, 128), lambda i: (i, 0)),
            pl.BlockSpec((_BK,), lambda i: (i,)),
        ],
        out_specs=pl.BlockSpec((1,), lambda i: (0,), memory_space=pltpu.SMEM),
        out_shape=jax.ShapeDtypeStruct((1,), jnp.float32),
        scratch_shapes=[pltpu.VMEM((8, 128), jnp.float32)],
    )(s2, mask.view(jnp.int8))
    return out[0]
